# baseline (speedup 1.0000x reference)
import functools

import jax
import jax.numpy as jnp
from jax import lax
from jax.experimental import pallas as pl
from jax.experimental.pallas import tpu as pltpu
from jax.experimental.pallas import tpu_sc as plsc

N = 10000
E = 320000
D = 128
NP = 10240
NW = 32
NLOC = NP // NW
CH = 6400
NCHUNK = E // CH
CAPL = CH // 16
G = 128
NCF = NCHUNK * 16

_mesh = plsc.VectorSubcoreMesh(core_axis_name="c", subcore_axis_name="s")

_i32 = jnp.int32
_f32 = jnp.float32


def _shift_idx(iota, k):
    return jnp.where(iota >= k, iota - k, iota + 16)


def _lane_splat_idx(iota, j):
    return iota * 0 + j


@functools.partial(
    pl.kernel,
    out_type=[
        jax.ShapeDtypeStruct((NW * NCHUNK * CH,), _i32),
        jax.ShapeDtypeStruct((NW * NCHUNK * CH,), _i32),
        jax.ShapeDtypeStruct((NW * NCF,), _i32),
    ],
    mesh=_mesh,
    compiler_params=pltpu.CompilerParams(needs_layout_passes=False),
    scratch_types=[
        pltpu.VMEM((CH,), _i32),
        pltpu.VMEM((CH,), _i32),
        pltpu.VMEM((CH + 16,), _i32),
        pltpu.VMEM((CH + 16,), _i32),
        pltpu.VMEM((CH + 16,), _i32),
        pltpu.VMEM((CH + 16,), _i32),
        pltpu.VMEM((NCF,), _i32),
        pltpu.VMEM((32,), _i32),
    ],
)
def _filter(srcv, dstv, slists, dlists, counts,
            dbuf, sbuf, dlane, slane, dlist, slist, cbuf, pbuf):
    c = lax.axis_index("c")
    s = lax.axis_index("s")
    wid = s * 2 + c
    lo = wid * NLOC

    zi = jnp.zeros((16,), _i32)
    iota = lax.iota(_i32, 16)
    lane_base = iota * CAPL

    def zinit(i, carry):
        dlane[pl.ds(i * 16, 16)] = zi
        slane[pl.ds(i * 16, 16)] = zi
        dlist[pl.ds(i * 16, 16)] = zi
        slist[pl.ds(i * 16, 16)] = zi
        return carry

    lax.fori_loop(0, CH // 16, zinit, 0)
    dlane[pl.ds(CH, 16)] = zi
    slane[pl.ds(CH, 16)] = zi
    dlist[pl.ds(CH, 16)] = zi
    slist[pl.ds(CH, 16)] = zi
    pbuf[pl.ds(0, 16)] = zi
    pbuf[pl.ds(16, 16)] = zi

    def chunk_body(ci, carry):
        pltpu.sync_copy(dstv.at[pl.ds(ci * CH, CH)], dbuf)
        pltpu.sync_copy(srcv.at[pl.ds(ci * CH, CH)], sbuf)

        def cgroup(g, percnt):
            d16 = dbuf[pl.ds(g * 16, 16)]
            s16 = sbuf[pl.ds(g * 16, 16)]
            m = (d16 >= lo) & (d16 < lo + NLOC)
            idx = jnp.where(m, lane_base + percnt, CH)
            plsc.store_scatter(dlane, [idx], d16 - lo)
            plsc.store_scatter(slane, [idx], s16)
            return percnt + m.astype(_i32)

        percnt = lax.fori_loop(0, CH // 16, cgroup, zi)

        inc = percnt
        mxv = percnt
        for k in (1, 2, 4, 8):
            pbuf[pl.ds(0, 16)] = inc
            inc = inc + plsc.load_gather(pbuf, [_shift_idx(iota, k)])
            pbuf[pl.ds(0, 16)] = mxv
            mxv = jnp.maximum(mxv, plsc.load_gather(pbuf, [_shift_idx(iota, k)]))
        excl = inc - percnt
        pbuf[pl.ds(0, 16)] = inc
        total_splat = plsc.load_gather(pbuf, [_lane_splat_idx(iota, 15)])
        pbuf[pl.ds(0, 16)] = mxv
        mx = plsc.load_gather(pbuf, [_lane_splat_idx(iota, 15)])[0]

        def mv(t, tv):
            src_idx = lane_base + tv
            sv = plsc.load_gather(slane, [src_idx])
            dv = plsc.load_gather(dlane, [src_idx])
            mm = tv < percnt
            di = jnp.where(mm, excl + tv, CH)
            plsc.store_scatter(slist, [di], sv)
            plsc.store_scatter(dlist, [di], dv)
            return tv + 1

        lax.fori_loop(0, mx, mv, zi)

        cbuf[pl.ds(ci * 16, 16)] = total_splat
        off = (wid * NCHUNK + ci) * CH
        pltpu.sync_copy(dlist.at[pl.ds(0, CH)], dlists.at[pl.ds(off, CH)])
        pltpu.sync_copy(slist.at[pl.ds(0, CH)], slists.at[pl.ds(off, CH)])
        return carry

    lax.fori_loop(0, NCHUNK, chunk_body, 0)
    pltpu.sync_copy(cbuf, counts.at[pl.ds(wid * NCF, NCF)])


@functools.partial(
    pl.kernel,
    out_type=jax.ShapeDtypeStruct((NP, D), _f32),
    mesh=_mesh,
    compiler_params=pltpu.CompilerParams(needs_layout_passes=False),
    scratch_types=[
        pltpu.VMEM((NCF,), _i32),
        pltpu.VMEM((CH + 16,), _i32),
        pltpu.VMEM((CH,), _i32),
        pltpu.VMEM((G,), _i32),
        pltpu.VMEM((G,), _i32),
        pltpu.VMEM((G, D), _f32),
        pltpu.VMEM((G, D), _f32),
        pltpu.VMEM((NLOC + 1, D), _f32),
        pltpu.VMEM((32,), _i32),
        pltpu.SemaphoreType.DMA,
        pltpu.SemaphoreType.DMA,
    ],
)
def _scatter_max(hp, slists, dlists, counts, out,
                 cbuf, dfrag, sfrag, sblk0, sblk1, rowbuf0, rowbuf1,
                 acc, pbuf, sem0, sem1):
    c = lax.axis_index("c")
    s = lax.axis_index("s")
    wid = s * 2 + c
    lo = wid * NLOC

    zf = jnp.zeros((16,), _f32)
    zi = jnp.zeros((16,), _i32)
    iota = lax.iota(_i32, 16)

    def zrow(r, carry):
        for j in range(D // 16):
            acc[r, pl.ds(j * 16, 16)] = zf
        return carry

    lax.fori_loop(0, NLOC + 1, zrow, 0)
    pbuf[pl.ds(0, 16)] = zi
    pbuf[pl.ds(16, 16)] = zi
    dfrag[pl.ds(CH, 16)] = zi

    pltpu.sync_copy(counts.at[pl.ds(wid * NCF, NCF)], cbuf)

    def chunk_body(ci, carry):
        cbase = (wid * NCHUNK + ci) * CH
        pltpu.sync_copy(dlists.at[pl.ds(cbase, CH)], dfrag.at[pl.ds(0, CH)])
        pltpu.sync_copy(slists.at[pl.ds(cbase, CH)], sfrag)
        cnt_splat = cbuf[pl.ds(ci * 16, 16)]
        cnt = cnt_splat[0]
        nb = (cnt + (G - 1)) // G
        nf = (nb + 1) // 2

        def process(bi, rowbuf, biv):
            ne = jnp.minimum(cnt - biv[0] * G, G)
            nev = jnp.minimum(cnt_splat - biv * G, G)
            ngrp = jnp.maximum(ne + 15, 0) // 16

            def group(gi, bgv):
                dv = dfrag[pl.ds(bi * G + gi * 16, 16)]
                dvm = jnp.where(bgv < nev, dv, NLOC)
                pbuf[pl.ds(0, 16)] = dvm
                ds_ = [plsc.load_gather(pbuf, [_lane_splat_idx(iota, j)])[0]
                       for j in range(16)]
                for j in range(16):
                    d = ds_[j]
                    e = gi * 16 + j
                    avs = [acc[d, pl.ds(jj * 16, 16)] for jj in range(D // 16)]
                    vvs = [rowbuf[e, pl.ds(jj * 16, 16)] for jj in range(D // 16)]
                    for jj in range(D // 16):
                        acc[d, pl.ds(jj * 16, 16)] = jnp.maximum(avs[jj], vvs[jj])
                return bgv + 16

            lax.fori_loop(0, ngrp, group, iota)
            return biv + 1

        def flight(fi, biv):
            b0 = 2 * fi
            b1 = 2 * fi + 1
            for j in range(G // 16):
                sblk0[pl.ds(j * 16, 16)] = sfrag[pl.ds(b0 * G + j * 16, 16)]
                sblk1[pl.ds(j * 16, 16)] = sfrag[pl.ds(b1 * G + j * 16, 16)]
            cp0 = pltpu.async_copy(hp.at[sblk0], rowbuf0, sem0)
            cp1 = pltpu.async_copy(hp.at[sblk1], rowbuf1, sem1)
            cp0.wait()
            biv = process(b0, rowbuf0, biv)
            cp1.wait()
            biv = process(b1, rowbuf1, biv)
            return biv

        lax.fori_loop(0, nf, flight, zi)
        return carry

    lax.fori_loop(0, NCHUNK, chunk_body, 0)
    pltpu.sync_copy(acc.at[pl.ds(0, NLOC)], out.at[pl.ds(lo, NLOC)])


BR = 1024


def _entry_body(x_ref, wp_ref, bp_ref, ws_ref, b_ref, hp_ref, xs_ref):
    x = x_ref[...]
    hp_ref[...] = jnp.maximum(
        jnp.dot(x, wp_ref[...], preferred_element_type=_f32)
        + bp_ref[...], 0.0)
    xs_ref[...] = (
        jnp.dot(x, ws_ref[...], preferred_element_type=_f32)
        + b_ref[...])


def _mid_body(xs_ref, agg_ref, wn_ref, wp_ref, bp_ref, ws_ref, b_ref,
              hp_ref, xs2_ref):
    t = xs_ref[...] + jnp.dot(agg_ref[...], wn_ref[...],
                              preferred_element_type=_f32)
    n = jnp.sqrt(jnp.sum(t * t, axis=1, keepdims=True))
    h = jnp.maximum(t / jnp.maximum(n, 1e-12), 0.0)
    hp_ref[...] = jnp.maximum(
        jnp.dot(h, wp_ref[...], preferred_element_type=_f32)
        + bp_ref[...], 0.0)
    xs2_ref[...] = (
        jnp.dot(h, ws_ref[...], preferred_element_type=_f32)
        + b_ref[...])


def _fin_body(xs_ref, agg_ref, wn_ref, out_ref):
    t = xs_ref[...] + jnp.dot(agg_ref[...], wn_ref[...],
                              preferred_element_type=_f32)
    n = jnp.sqrt(jnp.sum(t * t, axis=1, keepdims=True))
    out_ref[...] = jnp.maximum(t / jnp.maximum(n, 1e-12), 0.0)


_row_spec = pl.BlockSpec((BR, D), lambda i: (i, 0))
_w_spec = pl.BlockSpec((D, D), lambda i: (0, 0))
_b_spec = pl.BlockSpec((1, D), lambda i: (0, 0))
_fd = jax.ShapeDtypeStruct((NP, D), _f32)


def _tc_entry(x, wp, bp, ws, b):
    return pl.pallas_call(
        _entry_body,
        grid=(NP // BR,),
        in_specs=[_row_spec, _w_spec, _b_spec, _w_spec, _b_spec],
        out_specs=[_row_spec, _row_spec],
        out_shape=[_fd, _fd],
    )(x, wp, bp, ws, b)


def _tc_mid(xs, agg, wn, wp, bp, ws, b):
    return pl.pallas_call(
        _mid_body,
        grid=(NP // BR,),
        in_specs=[_row_spec, _row_spec, _w_spec, _w_spec, _b_spec,
                  _w_spec, _b_spec],
        out_specs=[_row_spec, _row_spec],
        out_shape=[_fd, _fd],
    )(xs, agg, wn, wp, bp, ws, b)


def _tc_fin(xs, agg, wn):
    return pl.pallas_call(
        _fin_body,
        grid=(NP // BR,),
        in_specs=[_row_spec, _row_spec, _w_spec],
        out_specs=_row_spec,
        out_shape=_fd,
    )(xs, agg, wn)


def kernel(inputs, edge_index, Wp1, bp1, Ws1, Wn1, b1, Wp2, bp2, Ws2, Wn2,
           b2, Wp3, bp3, Ws3, Wn3, b3):
    x = jnp.pad(inputs, ((0, NP - N), (0, 0)))
    src = edge_index[0].astype(_i32)
    dst = edge_index[1].astype(_i32)
    bp1r, b1r = bp1.reshape(1, D), b1.reshape(1, D)
    bp2r, b2r = bp2.reshape(1, D), b2.reshape(1, D)
    bp3r, b3r = bp3.reshape(1, D), b3.reshape(1, D)

    slists, dlists, counts = _filter(src, dst)

    hp1, xs1 = _tc_entry(x, Wp1, bp1r, Ws1, b1r)
    agg1 = _scatter_max(hp1, slists, dlists, counts)
    hp2, xs2 = _tc_mid(xs1, agg1, Wn1, Wp2, bp2r, Ws2, b2r)
    agg2 = _scatter_max(hp2, slists, dlists, counts)
    hp3, xs3 = _tc_mid(xs2, agg2, Wn2, Wp3, bp3r, Ws3, b3r)
    agg3 = _scatter_max(hp3, slists, dlists, counts)
    h = _tc_fin(xs3, agg3, Wn3)
    return h[:N]


# speedup vs baseline: 1.1921x; 1.1921x over previous
import functools

import jax
import jax.numpy as jnp
from jax import lax
from jax.experimental import pallas as pl
from jax.experimental.pallas import tpu as pltpu
from jax.experimental.pallas import tpu_sc as plsc

N = 10000
E = 320000
D = 128
NP = 10240
NW = 32
NLOC = NP // NW
CH = 6400
NCHUNK = E // CH
CAPL = CH // 16
G = 128
NCF = NCHUNK * 16

_mesh = plsc.VectorSubcoreMesh(core_axis_name="c", subcore_axis_name="s")

_i32 = jnp.int32
_f32 = jnp.float32


def _shift_idx(iota, k):
    return jnp.where(iota >= k, iota - k, iota + 16)


def _lane_splat_idx(iota, j):
    return iota * 0 + j


@functools.partial(
    pl.kernel,
    out_type=[
        jax.ShapeDtypeStruct((NW * NCHUNK * CH,), _i32),
        jax.ShapeDtypeStruct((NW * NCHUNK * CH,), _i32),
        jax.ShapeDtypeStruct((NW * NCF,), _i32),
    ],
    mesh=_mesh,
    compiler_params=pltpu.CompilerParams(needs_layout_passes=False),
    scratch_types=[
        pltpu.VMEM((CH,), _i32),
        pltpu.VMEM((CH,), _i32),
        pltpu.VMEM((CH + 16,), _i32),
        pltpu.VMEM((CH + 16,), _i32),
        pltpu.VMEM((CH + 16,), _i32),
        pltpu.VMEM((CH + 16,), _i32),
        pltpu.VMEM((NCF,), _i32),
        pltpu.VMEM((32,), _i32),
    ],
)
def _filter(srcv, dstv, slists, dlists, counts,
            dbuf, sbuf, dlane, slane, dlist, slist, cbuf, pbuf):
    c = lax.axis_index("c")
    s = lax.axis_index("s")
    wid = s * 2 + c
    lo = wid * NLOC

    zi = jnp.zeros((16,), _i32)
    iota = lax.iota(_i32, 16)
    lane_base = iota * CAPL

    def zinit(i, carry):
        dlane[pl.ds(i * 16, 16)] = zi
        slane[pl.ds(i * 16, 16)] = zi
        dlist[pl.ds(i * 16, 16)] = zi
        slist[pl.ds(i * 16, 16)] = zi
        return carry

    lax.fori_loop(0, CH // 16, zinit, 0)
    dlane[pl.ds(CH, 16)] = zi
    slane[pl.ds(CH, 16)] = zi
    dlist[pl.ds(CH, 16)] = zi
    slist[pl.ds(CH, 16)] = zi
    pbuf[pl.ds(0, 16)] = zi
    pbuf[pl.ds(16, 16)] = zi

    def chunk_body(ci, carry):
        pltpu.sync_copy(dstv.at[pl.ds(ci * CH, CH)], dbuf)
        pltpu.sync_copy(srcv.at[pl.ds(ci * CH, CH)], sbuf)

        def cgroup(g, percnt):
            d16 = dbuf[pl.ds(g * 16, 16)]
            s16 = sbuf[pl.ds(g * 16, 16)]
            m = (d16 >= lo) & (d16 < lo + NLOC)
            idx = jnp.where(m, lane_base + percnt, CH)
            plsc.store_scatter(dlane, [idx], d16 - lo)
            plsc.store_scatter(slane, [idx], s16)
            return percnt + m.astype(_i32)

        percnt = lax.fori_loop(0, CH // 16, cgroup, zi)

        inc = percnt
        mxv = percnt
        for k in (1, 2, 4, 8):
            pbuf[pl.ds(0, 16)] = inc
            inc = inc + plsc.load_gather(pbuf, [_shift_idx(iota, k)])
            pbuf[pl.ds(0, 16)] = mxv
            mxv = jnp.maximum(mxv, plsc.load_gather(pbuf, [_shift_idx(iota, k)]))
        excl = inc - percnt
        pbuf[pl.ds(0, 16)] = inc
        total_splat = plsc.load_gather(pbuf, [_lane_splat_idx(iota, 15)])
        pbuf[pl.ds(0, 16)] = mxv
        mx = plsc.load_gather(pbuf, [_lane_splat_idx(iota, 15)])[0]

        def mv(t, tv):
            src_idx = lane_base + tv
            sv = plsc.load_gather(slane, [src_idx])
            dv = plsc.load_gather(dlane, [src_idx])
            mm = tv < percnt
            di = jnp.where(mm, excl + tv, CH)
            plsc.store_scatter(slist, [di], sv)
            plsc.store_scatter(dlist, [di], dv)
            return tv + 1

        lax.fori_loop(0, mx, mv, zi)

        cbuf[pl.ds(ci * 16, 16)] = total_splat
        off = (wid * NCHUNK + ci) * CH
        pltpu.sync_copy(dlist.at[pl.ds(0, CH)], dlists.at[pl.ds(off, CH)])
        pltpu.sync_copy(slist.at[pl.ds(0, CH)], slists.at[pl.ds(off, CH)])
        return carry

    lax.fori_loop(0, NCHUNK, chunk_body, 0)
    pltpu.sync_copy(cbuf, counts.at[pl.ds(wid * NCF, NCF)])


@functools.partial(
    pl.kernel,
    out_type=jax.ShapeDtypeStruct((NP, D), _f32),
    mesh=_mesh,
    compiler_params=pltpu.CompilerParams(needs_layout_passes=False),
    scratch_types=[
        pltpu.VMEM((NCF,), _i32),
        pltpu.VMEM((CH + 16,), _i32),
        pltpu.VMEM((CH,), _i32),
        pltpu.VMEM((G,), _i32),
        pltpu.VMEM((G, D), _f32),
        pltpu.VMEM((NLOC + 1, D), _f32),
        pltpu.VMEM((32,), _i32),
        pltpu.SemaphoreType.DMA,
    ],
)
def _scatter_max(hp, slists, dlists, counts, out,
                 cbuf, dfrag, sfrag, sblk, rowbuf, acc, pbuf, sem):
    c = lax.axis_index("c")
    s = lax.axis_index("s")
    wid = s * 2 + c
    lo = wid * NLOC

    zf = jnp.zeros((16,), _f32)
    zi = jnp.zeros((16,), _i32)
    iota = lax.iota(_i32, 16)

    def zrow(r, carry):
        for j in range(D // 16):
            acc[r, pl.ds(j * 16, 16)] = zf
        return carry

    lax.fori_loop(0, NLOC + 1, zrow, 0)
    pbuf[pl.ds(0, 16)] = zi
    pbuf[pl.ds(16, 16)] = zi
    dfrag[pl.ds(CH, 16)] = zi

    pltpu.sync_copy(counts.at[pl.ds(wid * NCF, NCF)], cbuf)

    def chunk_body(ci, carry):
        cbase = (wid * NCHUNK + ci) * CH
        pltpu.sync_copy(dlists.at[pl.ds(cbase, CH)], dfrag.at[pl.ds(0, CH)])
        pltpu.sync_copy(slists.at[pl.ds(cbase, CH)], sfrag)
        cnt_splat = cbuf[pl.ds(ci * 16, 16)]
        cnt = cnt_splat[0]
        nb = (cnt + (G - 1)) // G

        def batch(bi, biv):
            for j in range(G // 16):
                sblk[pl.ds(j * 16, 16)] = sfrag[pl.ds(bi * G + j * 16, 16)]
            cp = pltpu.async_copy(hp.at[sblk], rowbuf, sem)
            ne = jnp.minimum(cnt - biv[0] * G, G)
            nev = jnp.minimum(cnt_splat - biv * G, G)
            ngrp = jnp.maximum(ne + 15, 0) // 16
            cp.wait()

            def group(gi, bgv):
                dv = dfrag[pl.ds(bi * G + gi * 16, 16)]
                dvm = jnp.where(bgv < nev, dv, NLOC)
                pbuf[pl.ds(0, 16)] = dvm
                ds_ = [plsc.load_gather(pbuf, [_lane_splat_idx(iota, j)])[0]
                       for j in range(16)]
                for j in range(16):
                    d = ds_[j]
                    e = gi * 16 + j
                    avs = [acc[d, pl.ds(jj * 16, 16)] for jj in range(D // 16)]
                    vvs = [rowbuf[e, pl.ds(jj * 16, 16)] for jj in range(D // 16)]
                    for jj in range(D // 16):
                        acc[d, pl.ds(jj * 16, 16)] = jnp.maximum(avs[jj], vvs[jj])
                return bgv + 16

            lax.fori_loop(0, ngrp, group, iota)
            return biv + 1

        lax.fori_loop(0, nb, batch, zi)
        return carry

    lax.fori_loop(0, NCHUNK, chunk_body, 0)
    pltpu.sync_copy(acc.at[pl.ds(0, NLOC)], out.at[pl.ds(lo, NLOC)])


BR = 1024


def _entry_body(x_ref, wp_ref, bp_ref, ws_ref, b_ref, hp_ref, xs_ref):
    x = x_ref[...]
    hp_ref[...] = jnp.maximum(
        jnp.dot(x, wp_ref[...], preferred_element_type=_f32)
        + bp_ref[...], 0.0)
    xs_ref[...] = (
        jnp.dot(x, ws_ref[...], preferred_element_type=_f32)
        + b_ref[...])


def _mid_body(xs_ref, agg_ref, wn_ref, wp_ref, bp_ref, ws_ref, b_ref,
              hp_ref, xs2_ref):
    t = xs_ref[...] + jnp.dot(agg_ref[...], wn_ref[...],
                              preferred_element_type=_f32)
    n = jnp.sqrt(jnp.sum(t * t, axis=1, keepdims=True))
    h = jnp.maximum(t / jnp.maximum(n, 1e-12), 0.0)
    hp_ref[...] = jnp.maximum(
        jnp.dot(h, wp_ref[...], preferred_element_type=_f32)
        + bp_ref[...], 0.0)
    xs2_ref[...] = (
        jnp.dot(h, ws_ref[...], preferred_element_type=_f32)
        + b_ref[...])


def _fin_body(xs_ref, agg_ref, wn_ref, out_ref):
    t = xs_ref[...] + jnp.dot(agg_ref[...], wn_ref[...],
                              preferred_element_type=_f32)
    n = jnp.sqrt(jnp.sum(t * t, axis=1, keepdims=True))
    out_ref[...] = jnp.maximum(t / jnp.maximum(n, 1e-12), 0.0)


_row_spec = pl.BlockSpec((BR, D), lambda i: (i, 0))
_w_spec = pl.BlockSpec((D, D), lambda i: (0, 0))
_b_spec = pl.BlockSpec((1, D), lambda i: (0, 0))
_fd = jax.ShapeDtypeStruct((NP, D), _f32)


def _tc_entry(x, wp, bp, ws, b):
    return pl.pallas_call(
        _entry_body,
        grid=(NP // BR,),
        in_specs=[_row_spec, _w_spec, _b_spec, _w_spec, _b_spec],
        out_specs=[_row_spec, _row_spec],
        out_shape=[_fd, _fd],
    )(x, wp, bp, ws, b)


def _tc_mid(xs, agg, wn, wp, bp, ws, b):
    return pl.pallas_call(
        _mid_body,
        grid=(NP // BR,),
        in_specs=[_row_spec, _row_spec, _w_spec, _w_spec, _b_spec,
                  _w_spec, _b_spec],
        out_specs=[_row_spec, _row_spec],
        out_shape=[_fd, _fd],
    )(xs, agg, wn, wp, bp, ws, b)


def _tc_fin(xs, agg, wn):
    return pl.pallas_call(
        _fin_body,
        grid=(NP // BR,),
        in_specs=[_row_spec, _row_spec, _w_spec],
        out_specs=_row_spec,
        out_shape=_fd,
    )(xs, agg, wn)


def kernel(inputs, edge_index, Wp1, bp1, Ws1, Wn1, b1, Wp2, bp2, Ws2, Wn2,
           b2, Wp3, bp3, Ws3, Wn3, b3):
    x = jnp.pad(inputs, ((0, NP - N), (0, 0)))
    src = edge_index[0].astype(_i32)
    dst = edge_index[1].astype(_i32)
    bp1r, b1r = bp1.reshape(1, D), b1.reshape(1, D)
    bp2r, b2r = bp2.reshape(1, D), b2.reshape(1, D)
    bp3r, b3r = bp3.reshape(1, D), b3.reshape(1, D)

    slists, dlists, counts = _filter(src, dst)

    hp1, xs1 = _tc_entry(x, Wp1, bp1r, Ws1, b1r)
    agg1 = _scatter_max(hp1, slists, dlists, counts)
    hp2, xs2 = _tc_mid(xs1, agg1, Wn1, Wp2, bp2r, Ws2, b2r)
    agg2 = _scatter_max(hp2, slists, dlists, counts)
    hp3, xs3 = _tc_mid(xs2, agg2, Wn2, Wp3, bp3r, Ws3, b3r)
    agg3 = _scatter_max(hp3, slists, dlists, counts)
    h = _tc_fin(xs3, agg3, Wn3)
    return h[:N]
